# baseline (speedup 1.0000x reference)
import jax
import jax.numpy as jnp
from jax.experimental import pallas as pl
from jax.experimental.pallas import tpu as pltpu


S = 2048
DH = 128
H = 16
HI = 4
DI = 64
TOPK = 512
BQ = 256
NQ = S // BQ
HALF = S // 2


def _bf16_round(x):
    u = jax.lax.bitcast_convert_type(x, jnp.int32)
    r = (u + 0x7FFF + ((u >> 16) & 1)) & ~0xFFFF
    return jax.lax.bitcast_convert_type(r, jnp.float32)


def _make_sel_kernel(width, row_off):
    def _sel_kernel(w_ref, iq_ref, ik_ref, tri_ref, mask_ref):
        i = pl.program_id(0)
        ik = ik_ref[...]
        acc = jnp.zeros((BQ, width), jnp.float32)
        for h in range(HI):
            lg = jax.lax.dot_general(
                iq_ref[h], ik, (((1,), (1,)), ((), ())),
                preferred_element_type=jnp.float32)
            rb = _bf16_round(jnp.maximum(lg, 0.0))
            acc = acc + rb * w_ref[h]

        rows = (row_off + i * BQ
                + jax.lax.broadcasted_iota(jnp.int32, (BQ, width), 0))
        cols = jax.lax.broadcasted_iota(jnp.int32, (BQ, width), 1)
        causal = cols <= rows
        si = jnp.where(
            causal,
            jnp.maximum(jax.lax.bitcast_convert_type(acc, jnp.int32), 0),
            -1)

        def vbody(_, st):
            lo, hi = st
            mid = lo + (hi - lo) // 2
            cnt = jnp.sum((si >= mid).astype(jnp.int32), axis=1,
                          keepdims=True)
            ge = cnt >= TOPK
            return jnp.where(ge, mid, lo), jnp.where(ge, hi, mid)

        lo0 = jnp.full((BQ, 1), -1, jnp.int32)
        hi0 = jnp.full((BQ, 1), 0x7F800000, jnp.int32)
        T, _ = jax.lax.fori_loop(0, 31, vbody, (lo0, hi0))

        n_gt = jnp.sum((si > T).astype(jnp.int32), axis=1, keepdims=True)
        need = (TOPK - n_gt).astype(jnp.float32)
        eq = si == T

        cum = jax.lax.dot_general(
            eq.astype(jnp.bfloat16), tri_ref[...],
            (((1,), (0,)), ((), ())),
            preferred_element_type=jnp.float32)

        mask = causal & ((si > T) | (eq & (cum <= need)))
        mask_ref[...] = jnp.where(mask, 0.0, -1e30).astype(jnp.float32)

    return _sel_kernel


def _attn_kernel(q_ref, k_ref, v_ref, mask_ref, o_ref):
    h = pl.program_id(1)
    qb = q_ref[0]
    kh = k_ref[h]
    vh = v_ref[h]
    lg = jax.lax.dot_general(
        qb, kh, (((1,), (1,)), ((), ())),
        preferred_element_type=jnp.float32)
    lg = lg + mask_ref[...]
    mx = jnp.max(lg, axis=1, keepdims=True)
    p = jnp.exp(lg - mx)
    s = jnp.sum(p, axis=1, keepdims=True)
    p = (p * (1.0 / s)).astype(jnp.bfloat16)
    o_ref[0] = jax.lax.dot_general(
        p, vh, (((1,), (0,)), ((), ())),
        preferred_element_type=jnp.float32)


def _sel_call(wr, iq_, ik_, tri, width, row_off, nq):
    return pl.pallas_call(
        _make_sel_kernel(width, row_off),
        grid=(nq,),
        in_specs=[
            pl.BlockSpec(memory_space=pltpu.SMEM),
            pl.BlockSpec((HI, BQ, DI),
                         lambda i: (0, i + row_off // BQ, 0)),
            pl.BlockSpec((width, DI), lambda i: (0, 0)),
            pl.BlockSpec((width, width), lambda i: (0, 0)),
        ],
        out_specs=pl.BlockSpec((BQ, width), lambda i: (i, 0)),
        out_shape=jax.ShapeDtypeStruct((nq * BQ, width), jnp.float32),
    )(wr, iq_, ik_[:width], tri)


def _attn_call(q_, k_, v_, mask, width, row_off, nq):
    return pl.pallas_call(
        _attn_kernel,
        grid=(nq, H),
        in_specs=[
            pl.BlockSpec((1, BQ, DH),
                         lambda i, h: (h, i + row_off // BQ, 0)),
            pl.BlockSpec((H, width, DH), lambda i, h: (0, 0, 0)),
            pl.BlockSpec((H, width, DH), lambda i, h: (0, 0, 0)),
            pl.BlockSpec((BQ, width), lambda i, h: (i, 0)),
        ],
        out_specs=pl.BlockSpec((1, BQ, DH), lambda i, h: (h, i, 0)),
        out_shape=jax.ShapeDtypeStruct((H, nq * BQ, DH), jnp.float32),
    )(q_, k_[:, :width], v_[:, :width], mask)


def kernel(q, k, v, iq, ik, w):
    bf = jnp.bfloat16
    q_ = (q[0] / jnp.sqrt(jnp.float32(DH))).astype(bf)
    k_ = k[0].astype(bf)
    v_ = v[0].astype(bf)
    iq_ = iq[0].astype(bf)
    ik_ = ik[0].astype(bf)
    wr = _bf16_round(w)

    ii = jnp.arange(S, dtype=jnp.int32)
    tri = (ii[:, None] <= ii[None, :]).astype(bf)

    mask_lo = _sel_call(wr, iq_, ik_, tri[:HALF, :HALF], HALF, 0, NQ // 2)
    mask_hi = _sel_call(wr, iq_, ik_, tri, S, HALF, NQ // 2)

    out_lo = _attn_call(q_, k_, v_, mask_lo, HALF, 0, NQ // 2)
    out_hi = _attn_call(q_, k_, v_, mask_hi, S, HALF, NQ // 2)

    return jnp.concatenate([out_lo, out_hi], axis=1)[None]
